# baseline (speedup 1.0000x reference)
import jax
import jax.numpy as jnp
from jax.experimental import pallas as pl
from jax.experimental.pallas import tpu as pltpu

_B = 4096
_IN = 768
_ED = 32
_K = 8192
_NL = 3
_NCAT = 18
_COMMIT_W = 0.25
_TB = 256
_G = _B // _TB
_F32 = jnp.float32


def _dot(a, b, dims):
    return jax.lax.dot_general(a, b, dimension_numbers=(dims, ((), ())),
                               preferred_element_type=_F32)


def _fused_kernel(x_ref,
                  ew0, eb0, ew1, eb1, ew2, eb2, ew3, eb3,
                  dw0, db0, dw1, db1, dw2, db2, dw3, db3,
                  cb_ref,
                  ids_ref, en_ref, part_ref):
    x = x_ref[...]

    h = jnp.maximum(_dot(x, ew0[...], ((1,), (0,))) + eb0[...], 0.0)
    h = jnp.maximum(_dot(h, ew1[...], ((1,), (0,))) + eb1[...], 0.0)
    h = jnp.maximum(_dot(h, ew2[...], ((1,), (0,))) + eb2[...], 0.0)
    h = _dot(h, ew3[...], ((1,), (0,))) + eb3[...]

    res = h
    z = jnp.zeros_like(h)
    ik = jax.lax.broadcasted_iota(jnp.int32, (_TB, _K), 1)
    ids_l = []
    en_l = []
    q_l = []
    for l in range(_NL):
        cb = cb_ref[l]
        scores = _dot(res, cb, ((1,), (1,)))
        rn = jnp.sum(res * res, axis=1, keepdims=True)
        cbn = jnp.sum(cb * cb, axis=1)[None, :]
        d = (rn - 2.0 * scores) + cbn
        m = jnp.min(d, axis=1, keepdims=True)
        ids = jnp.min(jnp.where(d == m, ik, _K), axis=1, keepdims=True)
        oh = (ik == ids).astype(_F32)
        emb = jax.lax.dot_general(oh, cb, dimension_numbers=(((1,), (0,)), ((), ())),
                                  preferred_element_type=_F32,
                                  precision=jax.lax.Precision.HIGHEST)
        dmin = jnp.sum((res - emb) ** 2, axis=1, keepdims=True)
        q_l.append(jnp.sum(dmin))
        en_l.append(jnp.sqrt(jnp.sum(emb * emb, axis=1, keepdims=True)))
        ids_l.append(ids)
        z = z + emb
        res = res - emb

    ids_ref[...] = jnp.concatenate(ids_l, axis=1)
    en_ref[...] = jnp.concatenate(en_l, axis=1)

    y = jnp.maximum(_dot(z, dw0[...], ((1,), (0,))) + db0[...], 0.0)
    y = jnp.maximum(_dot(y, dw1[...], ((1,), (0,))) + db1[...], 0.0)
    y = jnp.maximum(_dot(y, dw2[...], ((1,), (0,))) + db2[...], 0.0)
    y = _dot(y, dw3[...], ((1,), (0,))) + db3[...]

    lane = jax.lax.broadcasted_iota(jnp.int32, (_TB, _IN), 1)
    mcont = lane < (_IN - _NCAT)
    yc = jnp.where(mcont, y, 0.0)
    nrm = jnp.sqrt(jnp.sum(yc * yc, axis=1, keepdims=True))
    ycn = yc / (nrm + 1e-12)
    mse = jnp.sum(jnp.where(mcont, (ycn - x) ** 2, 0.0))
    cat = y
    tgt = x
    bce_e = jnp.maximum(cat, 0.0) - cat * tgt + jnp.log1p(jnp.exp(-jnp.abs(cat)))
    bce = jnp.sum(jnp.where(mcont, 0.0, bce_e))

    part_ref[0, 0, 0] = q_l[0]
    part_ref[0, 0, 1] = q_l[1]
    part_ref[0, 0, 2] = q_l[2]
    part_ref[0, 0, 3] = mse
    part_ref[0, 0, 4] = bce


def kernel(x, enc_params, dec_params, codebooks):
    biases2d = lambda ps: [p for W, b in ps for p in (W, b.reshape(1, -1))]
    ew = biases2d(enc_params)
    dw = biases2d(dec_params)

    const = lambda shape: pl.BlockSpec(shape, lambda i: (0,) * len(shape))
    in_specs = [pl.BlockSpec((_TB, _IN), lambda i: (i, 0))]
    for p in ew + dw:
        in_specs.append(const(p.shape))
    in_specs.append(const(codebooks.shape))

    out_shapes = (
        jax.ShapeDtypeStruct((_B, _NL), jnp.int32),
        jax.ShapeDtypeStruct((_B, _NL), _F32),
        jax.ShapeDtypeStruct((_G, 1, 8), _F32),
    )
    out_specs = (
        pl.BlockSpec((_TB, _NL), lambda i: (i, 0)),
        pl.BlockSpec((_TB, _NL), lambda i: (i, 0)),
        pl.BlockSpec((1, 1, 8), lambda i: (i, 0, 0), memory_space=pltpu.SMEM),
    )

    ids, en, part = pl.pallas_call(
        _fused_kernel,
        grid=(_G,),
        in_specs=in_specs,
        out_specs=out_specs,
        out_shape=out_shapes,
        compiler_params=pltpu.CompilerParams(
            dimension_semantics=("arbitrary",),
        ),
    )(x, *ew, *dw, codebooks)

    s = jnp.sum(part.reshape(_G, 8), axis=0)
    quantize_loss = (1.0 + _COMMIT_W) * (s[0] + s[1] + s[2]) / _B
    reconstruction_loss = (s[3] + s[4]) / _B
    loss = reconstruction_loss + quantize_loss
    embs_norm = en.T
    return loss, reconstruction_loss, quantize_loss, ids, embs_norm


# speedup vs baseline: 2.3090x; 2.3090x over previous
import jax
import jax.numpy as jnp
from jax.experimental import pallas as pl
from jax.experimental.pallas import tpu as pltpu

_B = 4096
_IN = 768
_ED = 32
_K = 8192
_NL = 3
_NCAT = 18
_COMMIT_W = 0.25
_TB = 256
_G = _B // _TB
_F32 = jnp.float32
_BF16 = jnp.bfloat16


def _dot(a, b, dims):
    return jax.lax.dot_general(a, b, dimension_numbers=(dims, ((), ())),
                               preferred_element_type=_F32)


def _fused_kernel(x_ref,
                  ew0, eb0, ew1, eb1, ew2, eb2, ew3, eb3,
                  dw0, db0, dw1, db1, dw2, db2, dw3, db3,
                  cb_ref,
                  ids_ref, en_ref, part_ref,
                  cbn_s, cbt_hi_s, cbt_mid_s, cbt_lo_s):
    @pl.when(pl.program_id(0) == 0)
    def _init():
        for l in range(_NL):
            cb = cb_ref[l]
            cbn_s[l] = jnp.sum(cb * cb, axis=1)[None, :]
            cbt = cb.T
            hi = cbt.astype(_BF16)
            r1 = cbt - hi.astype(_F32)
            mid = r1.astype(_BF16)
            lo = (r1 - mid.astype(_F32)).astype(_BF16)
            cbt_hi_s[l] = hi
            cbt_mid_s[l] = mid
            cbt_lo_s[l] = lo

    x = x_ref[...]

    h = jnp.maximum(_dot(x, ew0[...], ((1,), (0,))) + eb0[...], 0.0)
    h = jnp.maximum(_dot(h, ew1[...], ((1,), (0,))) + eb1[...], 0.0)
    h = jnp.maximum(_dot(h, ew2[...], ((1,), (0,))) + eb2[...], 0.0)
    h = _dot(h, ew3[...], ((1,), (0,))) + eb3[...]

    res = h
    z = jnp.zeros_like(h)
    ik = jax.lax.broadcasted_iota(jnp.int32, (_TB, _K), 1)
    isub = jax.lax.broadcasted_iota(jnp.int32, (_K, _TB), 0)
    ids_l = []
    en_l = []
    q_l = []
    for l in range(_NL):
        cb = cb_ref[l]
        scores = _dot(res, cb, ((1,), (1,)))
        rn = jnp.sum(res * res, axis=1, keepdims=True)
        d = (rn - 2.0 * scores) + cbn_s[l]
        m = jnp.min(d, axis=1, keepdims=True)
        ids = jnp.min(jnp.where(d == m, ik, _K), axis=1, keepdims=True)
        idsT = ids.reshape(1, _TB)
        ohT = (isub == idsT).astype(_BF16)
        embT = (_dot(cbt_hi_s[l], ohT, ((1,), (0,)))
                + _dot(cbt_mid_s[l], ohT, ((1,), (0,)))
                + _dot(cbt_lo_s[l], ohT, ((1,), (0,))))
        emb = embT.T
        dmin = jnp.sum((res - emb) ** 2, axis=1, keepdims=True)
        q_l.append(jnp.sum(dmin))
        en_l.append(jnp.sqrt(jnp.sum(emb * emb, axis=1, keepdims=True)))
        ids_l.append(ids)
        z = z + emb
        res = res - emb

    ids_ref[...] = jnp.concatenate(ids_l, axis=1)
    en_ref[...] = jnp.concatenate(en_l, axis=1)

    y = jnp.maximum(_dot(z, dw0[...], ((1,), (0,))) + db0[...], 0.0)
    y = jnp.maximum(_dot(y, dw1[...], ((1,), (0,))) + db1[...], 0.0)
    y = jnp.maximum(_dot(y, dw2[...], ((1,), (0,))) + db2[...], 0.0)
    y = _dot(y, dw3[...], ((1,), (0,))) + db3[...]

    lane = jax.lax.broadcasted_iota(jnp.int32, (_TB, _IN), 1)
    mcont = lane < (_IN - _NCAT)
    yc = jnp.where(mcont, y, 0.0)
    nrm = jnp.sqrt(jnp.sum(yc * yc, axis=1, keepdims=True))
    ycn = yc / (nrm + 1e-12)
    mse = jnp.sum(jnp.where(mcont, (ycn - x) ** 2, 0.0))
    cat = y
    tgt = x
    bce_e = jnp.maximum(cat, 0.0) - cat * tgt + jnp.log1p(jnp.exp(-jnp.abs(cat)))
    bce = jnp.sum(jnp.where(mcont, 0.0, bce_e))

    part_ref[0, 0, 0] = q_l[0]
    part_ref[0, 0, 1] = q_l[1]
    part_ref[0, 0, 2] = q_l[2]
    part_ref[0, 0, 3] = mse
    part_ref[0, 0, 4] = bce


def kernel(x, enc_params, dec_params, codebooks):
    biases2d = lambda ps: [p for W, b in ps for p in (W, b.reshape(1, -1))]
    ew = biases2d(enc_params)
    dw = biases2d(dec_params)

    const = lambda shape: pl.BlockSpec(shape, lambda i: (0,) * len(shape))
    in_specs = [pl.BlockSpec((_TB, _IN), lambda i: (i, 0))]
    for p in ew + dw:
        in_specs.append(const(p.shape))
    in_specs.append(const(codebooks.shape))

    out_shapes = (
        jax.ShapeDtypeStruct((_B, _NL), jnp.int32),
        jax.ShapeDtypeStruct((_B, _NL), _F32),
        jax.ShapeDtypeStruct((_G, 1, 8), _F32),
    )
    out_specs = (
        pl.BlockSpec((_TB, _NL), lambda i: (i, 0)),
        pl.BlockSpec((_TB, _NL), lambda i: (i, 0)),
        pl.BlockSpec((1, 1, 8), lambda i: (i, 0, 0), memory_space=pltpu.SMEM),
    )

    ids, en, part = pl.pallas_call(
        _fused_kernel,
        grid=(_G,),
        in_specs=in_specs,
        out_specs=out_specs,
        out_shape=out_shapes,
        scratch_shapes=[
            pltpu.VMEM((_NL, 1, _K), _F32),
            pltpu.VMEM((_NL, _ED, _K), _BF16),
            pltpu.VMEM((_NL, _ED, _K), _BF16),
            pltpu.VMEM((_NL, _ED, _K), _BF16),
        ],
        compiler_params=pltpu.CompilerParams(
            dimension_semantics=("arbitrary",),
        ),
    )(x, *ew, *dw, codebooks)

    s = jnp.sum(part.reshape(_G, 8), axis=0)
    quantize_loss = (1.0 + _COMMIT_W) * (s[0] + s[1] + s[2]) / _B
    reconstruction_loss = (s[3] + s[4]) / _B
    loss = reconstruction_loss + quantize_loss
    embs_norm = en.T
    return loss, reconstruction_loss, quantize_loss, ids, embs_norm


# speedup vs baseline: 2.3331x; 1.0104x over previous
import jax
import jax.numpy as jnp
from jax.experimental import pallas as pl
from jax.experimental.pallas import tpu as pltpu

_B = 4096
_IN = 768
_ED = 32
_K = 8192
_NL = 3
_NCAT = 18
_COMMIT_W = 0.25
_TB = 512
_G = _B // _TB
_F32 = jnp.float32
_BF16 = jnp.bfloat16


def _dot(a, b, dims):
    return jax.lax.dot_general(a, b, dimension_numbers=(dims, ((), ())),
                               preferred_element_type=_F32)


def _fused_kernel(x_ref,
                  ew0, eb0, ew1, eb1, ew2, eb2, ew3, eb3,
                  dw0, db0, dw1, db1, dw2, db2, dw3, db3,
                  cb_ref,
                  ids_ref, en_ref, part_ref,
                  cbn_s, cbt_s, cbt_hi_s, cbt_mid_s, cbt_lo_s):
    @pl.when(pl.program_id(0) == 0)
    def _init():
        for l in range(_NL):
            cb = cb_ref[l]
            cbn_s[l] = jnp.sum(cb * cb, axis=1)[None, :]
            cbt = cb.T
            cbt_s[l] = cbt
            hi = cbt.astype(_BF16)
            r1 = cbt - hi.astype(_F32)
            mid = r1.astype(_BF16)
            lo = (r1 - mid.astype(_F32)).astype(_BF16)
            cbt_hi_s[l] = hi
            cbt_mid_s[l] = mid
            cbt_lo_s[l] = lo

    x = x_ref[...]

    h = jnp.maximum(_dot(x, ew0[...], ((1,), (0,))) + eb0[...], 0.0)
    h = jnp.maximum(_dot(h, ew1[...], ((1,), (0,))) + eb1[...], 0.0)
    h = jnp.maximum(_dot(h, ew2[...], ((1,), (0,))) + eb2[...], 0.0)
    h = _dot(h, ew3[...], ((1,), (0,))) + eb3[...]

    res = h
    z = jnp.zeros_like(h)
    ik = jax.lax.broadcasted_iota(jnp.int32, (_TB, _K), 1)
    isub = jax.lax.broadcasted_iota(jnp.int32, (_K, _TB), 0)
    ids_l = []
    en_l = []
    q_l = []
    for l in range(_NL):
        scores = _dot(res, cbt_s[l], ((1,), (0,)))
        rn = jnp.sum(res * res, axis=1, keepdims=True)
        d = (rn - 2.0 * scores) + cbn_s[l]
        ids = jnp.argmin(d, axis=1).reshape(_TB, 1).astype(jnp.int32)
        idsT = ids.reshape(1, _TB)
        ohT = (isub == idsT).astype(_BF16)
        embT = (_dot(cbt_hi_s[l], ohT, ((1,), (0,)))
                + _dot(cbt_mid_s[l], ohT, ((1,), (0,)))
                + _dot(cbt_lo_s[l], ohT, ((1,), (0,))))
        emb = embT.T
        dmin = jnp.sum((res - emb) ** 2, axis=1, keepdims=True)
        q_l.append(jnp.sum(dmin))
        en_l.append(jnp.sqrt(jnp.sum(emb * emb, axis=1, keepdims=True)))
        ids_l.append(ids)
        z = z + emb
        res = res - emb

    ids_ref[...] = jnp.concatenate(ids_l, axis=1)
    en_ref[...] = jnp.concatenate(en_l, axis=1)

    y = jnp.maximum(_dot(z, dw0[...], ((1,), (0,))) + db0[...], 0.0)
    y = jnp.maximum(_dot(y, dw1[...], ((1,), (0,))) + db1[...], 0.0)
    y = jnp.maximum(_dot(y, dw2[...], ((1,), (0,))) + db2[...], 0.0)
    y = _dot(y, dw3[...], ((1,), (0,))) + db3[...]

    lane = jax.lax.broadcasted_iota(jnp.int32, (_TB, _IN), 1)
    mcont = lane < (_IN - _NCAT)
    yc = jnp.where(mcont, y, 0.0)
    nrm = jnp.sqrt(jnp.sum(yc * yc, axis=1, keepdims=True))
    ycn = yc / (nrm + 1e-12)
    mse = jnp.sum(jnp.where(mcont, (ycn - x) ** 2, 0.0))
    cat = y
    tgt = x
    bce_e = jnp.maximum(cat, 0.0) - cat * tgt + jnp.log1p(jnp.exp(-jnp.abs(cat)))
    bce = jnp.sum(jnp.where(mcont, 0.0, bce_e))

    part_ref[0, 0, 0] = q_l[0]
    part_ref[0, 0, 1] = q_l[1]
    part_ref[0, 0, 2] = q_l[2]
    part_ref[0, 0, 3] = mse
    part_ref[0, 0, 4] = bce


def kernel(x, enc_params, dec_params, codebooks):
    biases2d = lambda ps: [p for W, b in ps for p in (W, b.reshape(1, -1))]
    ew = biases2d(enc_params)
    dw = biases2d(dec_params)

    const = lambda shape: pl.BlockSpec(shape, lambda i: (0,) * len(shape))
    in_specs = [pl.BlockSpec((_TB, _IN), lambda i: (i, 0))]
    for p in ew + dw:
        in_specs.append(const(p.shape))
    in_specs.append(const(codebooks.shape))

    out_shapes = (
        jax.ShapeDtypeStruct((_B, _NL), jnp.int32),
        jax.ShapeDtypeStruct((_B, _NL), _F32),
        jax.ShapeDtypeStruct((_G, 1, 8), _F32),
    )
    out_specs = (
        pl.BlockSpec((_TB, _NL), lambda i: (i, 0)),
        pl.BlockSpec((_TB, _NL), lambda i: (i, 0)),
        pl.BlockSpec((1, 1, 8), lambda i: (i, 0, 0), memory_space=pltpu.SMEM),
    )

    ids, en, part = pl.pallas_call(
        _fused_kernel,
        grid=(_G,),
        in_specs=in_specs,
        out_specs=out_specs,
        out_shape=out_shapes,
        scratch_shapes=[
            pltpu.VMEM((_NL, 1, _K), _F32),
            pltpu.VMEM((_NL, _ED, _K), _F32),
            pltpu.VMEM((_NL, _ED, _K), _BF16),
            pltpu.VMEM((_NL, _ED, _K), _BF16),
            pltpu.VMEM((_NL, _ED, _K), _BF16),
        ],
        compiler_params=pltpu.CompilerParams(
            dimension_semantics=("arbitrary",),
        ),
    )(x, *ew, *dw, codebooks)

    s = jnp.sum(part.reshape(_G, 8), axis=0)
    quantize_loss = (1.0 + _COMMIT_W) * (s[0] + s[1] + s[2]) / _B
    reconstruction_loss = (s[3] + s[4]) / _B
    loss = reconstruction_loss + quantize_loss
    embs_norm = en.T
    return loss, reconstruction_loss, quantize_loss, ids, embs_norm


# speedup vs baseline: 2.3853x; 1.0224x over previous
import jax
import jax.numpy as jnp
from jax.experimental import pallas as pl
from jax.experimental.pallas import tpu as pltpu

_B = 4096
_IN = 768
_ED = 32
_K = 8192
_NL = 3
_NCAT = 18
_COMMIT_W = 0.25
_TB = 512
_G = _B // _TB
_F32 = jnp.float32
_BF16 = jnp.bfloat16


def _dot(a, b, dims):
    return jax.lax.dot_general(a, b, dimension_numbers=(dims, ((), ())),
                               preferred_element_type=_F32)


def _fused_kernel(x_ref,
                  ew0, eb0, ew1, eb1, ew2, eb2, ew3, eb3,
                  dw0, db0, dw1, db1, dw2, db2, dw3, db3,
                  cb_ref,
                  ids_ref, en_ref, part_ref,
                  cbn_s, cbt2_s, cbt_hi_s, cbt_mid_s, cbt_lo_s):
    @pl.when(pl.program_id(0) == 0)
    def _init():
        for l in range(_NL):
            cb = cb_ref[l]
            cbn_s[l] = jnp.sum(cb * cb, axis=1)[None, :]
            cbt = cb.T
            cbt2_s[l] = cbt + cbt
            hi = cbt.astype(_BF16)
            r1 = cbt - hi.astype(_F32)
            mid = r1.astype(_BF16)
            lo = (r1 - mid.astype(_F32)).astype(_BF16)
            cbt_hi_s[l] = hi
            cbt_mid_s[l] = mid
            cbt_lo_s[l] = lo

    x = x_ref[...]

    h = jnp.maximum(_dot(x, ew0[...], ((1,), (0,))) + eb0[...], 0.0)
    h = jnp.maximum(_dot(h, ew1[...], ((1,), (0,))) + eb1[...], 0.0)
    h = jnp.maximum(_dot(h, ew2[...], ((1,), (0,))) + eb2[...], 0.0)
    h = _dot(h, ew3[...], ((1,), (0,))) + eb3[...]

    res = h
    z = jnp.zeros_like(h)
    ik = jax.lax.broadcasted_iota(jnp.int32, (_TB, _K), 1)
    isub = jax.lax.broadcasted_iota(jnp.int32, (_K, _TB), 0)
    ids_l = []
    en_l = []
    q_l = []
    for l in range(_NL):
        scores2 = _dot(res, cbt2_s[l], ((1,), (0,)))
        rn = jnp.sum(res * res, axis=1, keepdims=True)
        d = (rn - scores2) + cbn_s[l]
        ids = jnp.argmin(d, axis=1).reshape(_TB, 1)
        idsT = ids.reshape(1, _TB)
        ohT = (isub == idsT).astype(_BF16)
        embT = (_dot(cbt_hi_s[l], ohT, ((1,), (0,)))
                + _dot(cbt_mid_s[l], ohT, ((1,), (0,)))
                + _dot(cbt_lo_s[l], ohT, ((1,), (0,))))
        emb = embT.T
        dmin = jnp.sum((res - emb) ** 2, axis=1, keepdims=True)
        q_l.append(jnp.sum(dmin))
        en_l.append(jnp.sqrt(jnp.sum(emb * emb, axis=1, keepdims=True)))
        ids_l.append(ids)
        z = z + emb
        res = res - emb

    ids_ref[...] = jnp.concatenate(ids_l, axis=1)
    en_ref[...] = jnp.concatenate(en_l, axis=1)

    y = jnp.maximum(_dot(z, dw0[...], ((1,), (0,))) + db0[...], 0.0)
    y = jnp.maximum(_dot(y, dw1[...], ((1,), (0,))) + db1[...], 0.0)
    y = jnp.maximum(_dot(y, dw2[...], ((1,), (0,))) + db2[...], 0.0)
    y = _dot(y, dw3[...], ((1,), (0,))) + db3[...]

    lane = jax.lax.broadcasted_iota(jnp.int32, (_TB, _IN), 1)
    mcont = lane < (_IN - _NCAT)
    yc = jnp.where(mcont, y, 0.0)
    nrm = jnp.sqrt(jnp.sum(yc * yc, axis=1, keepdims=True))
    ycn = yc / (nrm + 1e-12)
    mse = jnp.sum(jnp.where(mcont, (ycn - x) ** 2, 0.0))
    cat = y
    tgt = x
    bce_e = jnp.maximum(cat, 0.0) - cat * tgt + jnp.log1p(jnp.exp(-jnp.abs(cat)))
    bce = jnp.sum(jnp.where(mcont, 0.0, bce_e))

    part_ref[0, 0, 0] = q_l[0]
    part_ref[0, 0, 1] = q_l[1]
    part_ref[0, 0, 2] = q_l[2]
    part_ref[0, 0, 3] = mse
    part_ref[0, 0, 4] = bce


def kernel(x, enc_params, dec_params, codebooks):
    biases2d = lambda ps: [p for W, b in ps for p in (W, b.reshape(1, -1))]
    ew = biases2d(enc_params)
    dw = biases2d(dec_params)

    const = lambda shape: pl.BlockSpec(shape, lambda i: (0,) * len(shape))
    in_specs = [pl.BlockSpec((_TB, _IN), lambda i: (i, 0))]
    for p in ew + dw:
        in_specs.append(const(p.shape))
    in_specs.append(const(codebooks.shape))

    out_shapes = (
        jax.ShapeDtypeStruct((_B, _NL), jnp.int32),
        jax.ShapeDtypeStruct((_B, _NL), _F32),
        jax.ShapeDtypeStruct((_G, 1, 8), _F32),
    )
    out_specs = (
        pl.BlockSpec((_TB, _NL), lambda i: (i, 0)),
        pl.BlockSpec((_TB, _NL), lambda i: (i, 0)),
        pl.BlockSpec((1, 1, 8), lambda i: (i, 0, 0), memory_space=pltpu.SMEM),
    )

    ids, en, part = pl.pallas_call(
        _fused_kernel,
        grid=(_G,),
        in_specs=in_specs,
        out_specs=out_specs,
        out_shape=out_shapes,
        scratch_shapes=[
            pltpu.VMEM((_NL, 1, _K), _F32),
            pltpu.VMEM((_NL, _ED, _K), _F32),
            pltpu.VMEM((_NL, _ED, _K), _BF16),
            pltpu.VMEM((_NL, _ED, _K), _BF16),
            pltpu.VMEM((_NL, _ED, _K), _BF16),
        ],
        compiler_params=pltpu.CompilerParams(
            dimension_semantics=("arbitrary",),
        ),
    )(x, *ew, *dw, codebooks)

    s = jnp.sum(part.reshape(_G, 8), axis=0)
    quantize_loss = (1.0 + _COMMIT_W) * (s[0] + s[1] + s[2]) / _B
    reconstruction_loss = (s[3] + s[4]) / _B
    loss = reconstruction_loss + quantize_loss
    embs_norm = en.T
    return loss, reconstruction_loss, quantize_loss, ids, embs_norm
